# Initial kernel scaffold; baseline (speedup 1.0000x reference)
import jax
import jax.numpy as jnp
from jax.experimental import pallas as pl


def kernel(pred, targets, grids):
    raise NotImplementedError("write your pallas kernel here")


# speedup vs baseline: 70.2486x; 70.2486x over previous
import functools
import math

import jax
import jax.numpy as jnp
from jax.experimental import pallas as pl
from jax.experimental.pallas import tpu as pltpu

_B = 16
_T = 8
_NC = 30
_N = 2100
_NT = _B * _T
_P = _T * 3


def _sp(x):
    return jnp.maximum(x, 0.0) + jnp.log1p(jnp.exp(-jnp.abs(x)))


_ATAN_C = (0.9999994160035323, -0.3333022235532034, 0.19951110891900398,
           -0.139332293932798, 0.0970935073714827, -0.05688089274199308,
           0.022566826126643333, -0.004257409078054553)


def _atan(x):
    t = jnp.abs(x)
    inv = t > 1.0
    z = jnp.where(inv, 1.0 / jnp.maximum(t, 1e-30), t)
    u = z * z
    p = jnp.float32(_ATAN_C[-1])
    for c in _ATAN_C[-2::-1]:
        p = p * u + jnp.float32(c)
    r = z * p
    r = jnp.where(inv, jnp.float32(math.pi / 2) - r, r)
    return jnp.sign(x) * r


def _loss_kernel(pred_ref, tx_ref, ty_ref, gx_ref, gy_ref, tb_ref, cls_ref,
                 out_ref):
    f32 = jnp.float32
    tx = tx_ref[...]
    ty = ty_ref[...]
    gx = gx_ref[...]
    gy = gy_ref[...]
    dxx = tx - gx
    dyy = ty - gy
    d = jnp.sqrt(dxx * dxx + dyy * dyy)
    iota = jax.lax.broadcasted_iota(jnp.int32, (_NT, _N), 1)
    big = jnp.int32(1 << 30)
    cs = []
    for _ in range(3):
        m = jnp.min(d, axis=1, keepdims=True)
        idx = jnp.min(jnp.where(d == m, iota, big), axis=1, keepdims=True)
        cs.append(idx)
        d = jnp.where(iota == idx, jnp.float32(jnp.inf), d)

    iota_pn = jax.lax.broadcasted_iota(jnp.int32, (_P, _N), 1)
    iota_pc = jax.lax.broadcasted_iota(jnp.int32, (_P, _NC), 1)
    later = (jax.lax.broadcasted_iota(jnp.int32, (_P, _P), 1)
             < jax.lax.broadcasted_iota(jnp.int32, (_P, _P), 0))
    later_f = later.astype(f32)

    sum_sp = f32(0.0)
    obj_corr = f32(0.0)
    cls_sum = f32(0.0)
    iou_sum = f32(0.0)
    m_cnt = f32(0.0)

    for i in range(_B):
        pim = pred_ref[i]
        sum_sp = sum_sp + jnp.sum(_sp(pim[4:5, :]))

        s = slice(i * _T, (i + 1) * _T)
        cells_i = jnp.concatenate([cs[0][s], cs[1][s], cs[2][s]], axis=0)
        oh = (cells_i == iota_pn).astype(f32)
        g = jax.lax.dot_general(oh, pim, (((1,), (1,)), ((), ())),
                                preferred_element_type=f32)

        tbox = tb_ref[i]
        b1x, b1y, b1w, b1h = (g[:, 0:1], g[:, 1:2], g[:, 2:3], g[:, 3:4])
        b2x, b2y, b2w, b2h = (tbox[:, 0:1], tbox[:, 1:2], tbox[:, 2:3],
                              tbox[:, 3:4])
        b1x1, b1x2 = b1x - b1w / 2, b1x + b1w / 2
        b1y1, b1y2 = b1y - b1h / 2, b1y + b1h / 2
        b2x1, b2x2 = b2x - b2w / 2, b2x + b2w / 2
        b2y1, b2y2 = b2y - b2h / 2, b2y + b2h / 2
        inter = (jnp.clip(jnp.minimum(b1x2, b2x2) - jnp.maximum(b1x1, b2x1),
                          0.0, None)
                 * jnp.clip(jnp.minimum(b1y2, b2y2) - jnp.maximum(b1y1, b2y1),
                            0.0, None))
        union = b1w * b1h + b2w * b2h - inter + 1e-07
        iou = inter / union
        cw = jnp.maximum(b1x2, b2x2) - jnp.minimum(b1x1, b2x1)
        ch = jnp.maximum(b1y2, b2y2) - jnp.minimum(b1y1, b2y1)
        c2 = cw * cw + ch * ch + 1e-07
        rho2 = ((b1x1 + b1x2 - b2x1 - b2x2) ** 2
                + (b1y1 + b1y2 - b2y1 - b2y2) ** 2) / 4
        v = (4.0 / math.pi ** 2) * (_atan(b1w / (b1h + 1e-07))
                                    - _atan(b2w / (b2h + 1e-07))) ** 2
        alpha = v / (1.0 - iou + v + 1e-07)
        ciou = iou - (rho2 / c2 + v * alpha)
        iou_sum = iou_sum + jnp.sum(1.0 - ciou)

        eq_cell = jax.lax.dot_general(oh, oh, (((1,), (1,)), ((), ())),
                                      preferred_element_type=f32)
        clsv = cls_ref[i]
        ohc = (clsv == iota_pc).astype(f32)
        eq_cls = jax.lax.dot_general(ohc, ohc, (((1,), (1,)), ((), ())),
                                     preferred_element_type=f32)
        udup = jnp.sum(eq_cell * later_f, axis=1, keepdims=True)
        ukeep = (udup == 0.0).astype(f32)
        cdup = jnp.sum(eq_cell * eq_cls * later_f, axis=1, keepdims=True)
        ckeep = (cdup == 0.0).astype(f32)

        m_cnt = m_cnt + jnp.sum(ukeep)

        pobj_g = g[:, 4:5]
        obj_corr = obj_corr + jnp.sum(ukeep * (5.0 * _sp(-pobj_g)
                                               - _sp(pobj_g)))

        pcls = g[:, 5:5 + _NC]
        spm = _sp(-pcls)
        spp = _sp(pcls)
        base = jnp.sum(0.05 * spm + 0.95 * spp, axis=1, keepdims=True)
        win = jnp.sum(ohc * (spm - spp), axis=1, keepdims=True)
        cls_sum = cls_sum + jnp.sum(ukeep * base) + 0.9 * jnp.sum(ckeep * win)

    loss_obj = (sum_sp + obj_corr) / f32(_B * _N)
    loss_cls = cls_sum / (m_cnt * _NC + 1e-12)
    out_ref[0, 0] = 10.0 * iou_sum / f32(_NT * 3) + loss_obj + loss_cls


@jax.jit
def kernel(pred, targets, grids):
    tx = targets[:, :, 1].reshape(_NT, 1)
    ty = targets[:, :, 2].reshape(_NT, 1)
    gx = grids[:, 0].reshape(1, _N)
    gy = grids[:, 1].reshape(1, _N)
    tb = jnp.tile(targets[:, :, 1:5], (1, 3, 1))
    cls = jnp.tile(targets[:, :, 0:1].astype(jnp.int32), (1, 3, 1))

    out = pl.pallas_call(
        _loss_kernel,
        out_shape=jax.ShapeDtypeStruct((1, 1), jnp.float32),
        out_specs=pl.BlockSpec(memory_space=pltpu.SMEM),
    )(pred, tx, ty, gx, gy, tb, cls)
    return out[0, 0]
